# baseline (speedup 1.0000x reference)
import functools

import jax
import jax.numpy as jnp
from jax import lax
from jax.experimental import pallas as pl
from jax.experimental.pallas import tpu as pltpu
from jax.experimental.pallas import tpu_sc as plsc

D_MODEL = 768
N_EXP = 8
RANK = 16
HID = N_EXP * RANK


def _stage_a(x_ref, wr_ref, br_ref, wd_ref, bd_ref, logit_ref, h_ref):
    x = x_ref[...]
    lt = lax.dot_general(wr_ref[...], x, (((0,), (1,)), ((), ())),
                         preferred_element_type=jnp.float32)
    logit_ref[...] = lt + br_ref[...]
    h = jnp.dot(x, wd_ref[...], preferred_element_type=jnp.float32)
    h_ref[...] = jnp.maximum(h + bd_ref[...], 0.0)


def _stage_c(x_ref, h_ref, w_ref, wuc_ref, out_ref):
    x = x_ref[...]
    w8t = w_ref[...]
    row = lax.broadcasted_iota(jnp.int32, (N_EXP, HID), 0)
    col = lax.broadcasted_iota(jnp.int32, (N_EXP, HID), 1)
    expand = (col // RANK == row).astype(jnp.float32)
    w128 = lax.dot_general(w8t, expand, (((0,), (0,)), ((), ())),
                           preferred_element_type=jnp.float32)
    hw = jnp.concatenate([h_ref[...] * w128, w128], axis=1)
    out_ref[...] = x + jnp.dot(hw, wuc_ref[...],
                               preferred_element_type=jnp.float32)


def _make_sc_gating(n_tok):
    info = plsc.get_sparse_core_info()
    nw = info.num_cores * info.num_subcores
    chunk = n_tok // nw
    flat = chunk * N_EXP
    mesh = plsc.VectorSubcoreMesh(core_axis_name="c", subcore_axis_name="s")

    @functools.partial(
        pl.kernel, mesh=mesh,
        out_type=jax.ShapeDtypeStruct((N_EXP * n_tok,), jnp.float32),
        scratch_types=[
            pltpu.VMEM((flat,), jnp.float32),
            pltpu.VMEM((flat,), jnp.float32),
        ],
    )
    def gate(logit_hbm, w_hbm, lv, wv):
        wid = lax.axis_index("s") * info.num_cores + lax.axis_index("c")
        base = wid * chunk
        for e in range(N_EXP):
            pltpu.sync_copy(logit_hbm.at[pl.ds(e * n_tok + base, chunk)],
                            lv.at[pl.ds(e * chunk, chunk)])

        def body(i, carry):
            off = i * 16
            g = [lv[pl.ds(e * chunk + off, 16)] for e in range(N_EXP)]
            v1 = g[0]
            for e in range(1, N_EXP):
                v1 = jnp.maximum(v1, g[e])
            neg = jnp.float32(-1e30)
            v2 = jnp.where(g[0] == v1, neg, g[0])
            for e in range(1, N_EXP):
                v2 = jnp.maximum(v2, jnp.where(g[e] == v1, neg, g[e]))
            s = jnp.exp(v2 - v1)
            wtop = 1.0 / (1.0 + s)
            wsec = 1.0 - wtop
            zero = jnp.zeros((16,), jnp.float32)
            for e in range(N_EXP):
                we = jnp.where(g[e] == v1, wtop,
                               jnp.where(g[e] == v2, wsec, zero))
                wv[pl.ds(e * chunk + off, 16)] = we
            return carry

        lax.fori_loop(0, chunk // 16, body, 0)
        for e in range(N_EXP):
            pltpu.sync_copy(wv.at[pl.ds(e * chunk, chunk)],
                            w_hbm.at[pl.ds(e * n_tok + base, chunk)])

    return gate


@jax.jit
def _hybrid(x, w_r, b_r, wd_cat, bd_cat, wu_full):
    n_tok = x.shape[0]
    tile = 4096
    grid = (n_tok // tile,)
    logits_t, h = pl.pallas_call(
        _stage_a,
        grid=grid,
        in_specs=[
            pl.BlockSpec((tile, D_MODEL), lambda i: (i, 0)),
            pl.BlockSpec((D_MODEL, N_EXP), lambda i: (0, 0)),
            pl.BlockSpec((N_EXP, 1), lambda i: (0, 0)),
            pl.BlockSpec((D_MODEL, HID), lambda i: (0, 0)),
            pl.BlockSpec((1, HID), lambda i: (0, 0)),
        ],
        out_specs=[
            pl.BlockSpec((N_EXP, tile), lambda i: (0, i)),
            pl.BlockSpec((tile, HID), lambda i: (i, 0)),
        ],
        out_shape=[
            jax.ShapeDtypeStruct((N_EXP, n_tok), jnp.float32),
            jax.ShapeDtypeStruct((n_tok, HID), jnp.float32),
        ],
        compiler_params=pltpu.CompilerParams(
            dimension_semantics=("arbitrary",),
        ),
    )(x, w_r, b_r.reshape(N_EXP, 1), wd_cat, bd_cat)

    w_flat = _make_sc_gating(n_tok)(logits_t.reshape(-1))
    w8t = w_flat.reshape(N_EXP, n_tok)

    return pl.pallas_call(
        _stage_c,
        grid=grid,
        in_specs=[
            pl.BlockSpec((tile, D_MODEL), lambda i: (i, 0)),
            pl.BlockSpec((tile, HID), lambda i: (i, 0)),
            pl.BlockSpec((N_EXP, tile), lambda i: (0, i)),
            pl.BlockSpec((2 * HID, D_MODEL), lambda i: (0, 0)),
        ],
        out_specs=pl.BlockSpec((tile, D_MODEL), lambda i: (i, 0)),
        out_shape=jax.ShapeDtypeStruct((n_tok, D_MODEL), jnp.float32),
        compiler_params=pltpu.CompilerParams(
            dimension_semantics=("arbitrary",),
        ),
    )(x, h, w8t, wu_full)


def kernel(x, W_r, b_r, W_down, b_down, W_up, b_up):
    wd_cat = jnp.transpose(W_down, (1, 0, 2)).reshape(D_MODEL, HID)
    bd_cat = b_down.reshape(1, HID)
    wu_cat = W_up.reshape(HID, D_MODEL)
    bu_rep = jnp.repeat(b_up, RANK, axis=0) / RANK
    wu_full = jnp.concatenate([wu_cat, bu_rep], axis=0)
    return _hybrid(x, W_r, b_r, wd_cat, bd_cat, wu_full)
